# Initial kernel scaffold; baseline (speedup 1.0000x reference)
import jax
import jax.numpy as jnp
from jax.experimental import pallas as pl


def kernel(x, W1, g1, b1, W2, g2, b2, W3, g3, b3, W4, g4, b4, W5, g5, b5, Ws, lng, lnb, Wl1, bl1, g6, b6, Wl2, bl2, g7, b7, Wl3, bl3):
    raise NotImplementedError("write your pallas kernel here")


# speedup vs baseline: 20.1203x; 20.1203x over previous
import functools
import math

import jax
import jax.numpy as jnp
from jax.experimental import pallas as pl
from jax.experimental.pallas import tpu as pltpu

_B, _N, _K, _EMB, _HEADS = 8, 1024, 20, 1024, 4
_EPS = 1e-5
_RB = 256
_NEG = float("-inf")
_SLOPE = 0.2


def _leaky(v):
    return jnp.where(v >= 0, v, _SLOPE * v)


def _attn_conv_body(xf_ref, xb_ref, wq_ref, wf_ref, g_ref, b_ref, out_ref):
    X = xf_ref[0]
    Xb = xb_ref[0]
    c = X.shape[0]

    G = jax.lax.dot_general(Xb, X, (((0,), (0,)), ((), ())),
                            preferred_element_type=jnp.float32)
    xx_row = jnp.sum(X * X, axis=0, keepdims=True)
    ones_c = jnp.ones((c, 1), jnp.float32)
    xx_col = jax.lax.dot_general(Xb * Xb, ones_c, (((0,), (0,)), ((), ())),
                                 preferred_element_type=jnp.float32)
    pd = 2.0 * G - xx_col - xx_row

    pdm = pd
    member = None
    for i in range(_K + 1):
        rowmax = jnp.max(pdm, axis=1, keepdims=True)
        sel = pdm == rowmax
        if i == 1:
            member = sel
        elif i > 1:
            member = jnp.logical_or(member, sel)
        if i < _K:
            pdm = jnp.where(sel, _NEG, pdm)

    gm = jnp.where(member, G, _NEG)
    m = jnp.max(gm, axis=1, keepdims=True)
    e = jnp.where(member, jnp.exp(G - m), 0.0)
    Z = jnp.sum(e, axis=1, keepdims=True)
    Wd = e / Z

    FT = jax.lax.dot_general(X, Wd, (((1,), (1,)), ((), ())),
                             preferred_element_type=jnp.float32)
    val = FT - Xb

    Y = jax.lax.dot_general(wq_ref[...], Xb, (((1,), (0,)), ((), ())),
                            preferred_element_type=jnp.float32)
    Y = Y + jax.lax.dot_general(wf_ref[...], val, (((1,), (0,)), ((), ())),
                                preferred_element_type=jnp.float32)
    Y = Y / jnp.sqrt(jnp.float32(1.0 + _EPS))
    Y = Y * g_ref[...] + b_ref[...]
    out_ref[0] = _leaky(Y)


def _attn_conv(x, wq, wf, g, b):
    Bn, c, n = x.shape
    oc = wq.shape[0]
    grid = (Bn, n // _RB)
    return pl.pallas_call(
        _attn_conv_body,
        grid=grid,
        in_specs=[
            pl.BlockSpec((1, c, n), lambda bi, r: (bi, 0, 0)),
            pl.BlockSpec((1, c, _RB), lambda bi, r: (bi, 0, r)),
            pl.BlockSpec((oc, c), lambda bi, r: (0, 0)),
            pl.BlockSpec((oc, c), lambda bi, r: (0, 0)),
            pl.BlockSpec((oc, 1), lambda bi, r: (0, 0)),
            pl.BlockSpec((oc, 1), lambda bi, r: (0, 0)),
        ],
        out_specs=pl.BlockSpec((1, oc, _RB), lambda bi, r: (bi, 0, r)),
        out_shape=jax.ShapeDtypeStruct((Bn, oc, n), jnp.float32),
        compiler_params=pltpu.CompilerParams(
            dimension_semantics=("parallel", "parallel")),
    )(x, x, wq, wf, g, b)


def _head_body(x1_ref, x2_ref, x3_ref, x4_ref, w5_ref, g5_ref, b5_ref,
               ws_ref, ap_ref):
    W5 = w5_ref[...]
    h = jax.lax.dot_general(W5[:, 0:64], x1_ref[0], (((1,), (0,)), ((), ())),
                            preferred_element_type=jnp.float32)
    h = h + jax.lax.dot_general(W5[:, 64:128], x2_ref[0],
                                (((1,), (0,)), ((), ())),
                                preferred_element_type=jnp.float32)
    h = h + jax.lax.dot_general(W5[:, 128:256], x3_ref[0],
                                (((1,), (0,)), ((), ())),
                                preferred_element_type=jnp.float32)
    h = h + jax.lax.dot_general(W5[:, 256:512], x4_ref[0],
                                (((1,), (0,)), ((), ())),
                                preferred_element_type=jnp.float32)
    h = h / jnp.sqrt(jnp.float32(1.0 + _EPS))
    h = _leaky(h * g5_ref[...] + b5_ref[...])
    S = _leaky(jax.lax.dot_general(ws_ref[...], h, (((1,), (0,)), ((), ())),
                                   preferred_element_type=jnp.float32))
    ap_ref[0] = jax.lax.dot_general(S, h, (((1,), (1,)), ((), ())),
                                    preferred_element_type=jnp.float32)


def _head(x1, x2, x3, x4, W5, g5, b5, Ws):
    return pl.pallas_call(
        _head_body,
        grid=(_B,),
        in_specs=[
            pl.BlockSpec((1, 64, _N), lambda bi: (bi, 0, 0)),
            pl.BlockSpec((1, 64, _N), lambda bi: (bi, 0, 0)),
            pl.BlockSpec((1, 128, _N), lambda bi: (bi, 0, 0)),
            pl.BlockSpec((1, 256, _N), lambda bi: (bi, 0, 0)),
            pl.BlockSpec((_EMB, 512), lambda bi: (0, 0)),
            pl.BlockSpec((_EMB, 1), lambda bi: (0, 0)),
            pl.BlockSpec((_EMB, 1), lambda bi: (0, 0)),
            pl.BlockSpec((_HEADS, _EMB), lambda bi: (0, 0)),
        ],
        out_specs=pl.BlockSpec((1, _HEADS, _EMB), lambda bi: (bi, 0, 0)),
        out_shape=jax.ShapeDtypeStruct((_B, _HEADS, _EMB), jnp.float32),
        compiler_params=pltpu.CompilerParams(
            dimension_semantics=("parallel",)),
    )(x1, x2, x3, x4, W5, g5, b5, Ws)


def _mlp_body(ap_ref, lng_ref, lnb_ref, wl1_ref, bl1_ref, g6_ref, b6_ref,
              wl2_ref, bl2_ref, g7_ref, b7_ref, wl3_ref, bl3_ref, out_ref):
    ap = ap_ref[...]
    mu = jnp.mean(ap, axis=1, keepdims=True)
    d = ap - mu
    var = jnp.mean(d * d, axis=1, keepdims=True)
    ap = d / jnp.sqrt(var + _EPS) * lng_ref[...] + lnb_ref[...]
    ap = _leaky(ap)
    sq = jnp.sqrt(jnp.float32(1.0 + _EPS))
    y = jax.lax.dot_general(ap, wl1_ref[...], (((1,), (1,)), ((), ())),
                            preferred_element_type=jnp.float32) + bl1_ref[...]
    y = _leaky(y / sq * g6_ref[...] + b6_ref[...])
    y = jax.lax.dot_general(y, wl2_ref[...], (((1,), (1,)), ((), ())),
                            preferred_element_type=jnp.float32) + bl2_ref[...]
    y = _leaky(y / sq * g7_ref[...] + b7_ref[...])
    out_ref[...] = jax.lax.dot_general(
        y, wl3_ref[...], (((1,), (1,)), ((), ())),
        preferred_element_type=jnp.float32) + bl3_ref[...]


def _mlp(ap, lng, lnb, Wl1, bl1, g6, b6, Wl2, bl2, g7, b7, Wl3, bl3):
    return pl.pallas_call(
        _mlp_body,
        out_shape=jax.ShapeDtypeStruct((_B, 40), jnp.float32),
    )(ap, lng, lnb, Wl1, bl1, g6, b6, Wl2, bl2, g7, b7, Wl3, bl3)


def kernel(x, W1, g1, b1, W2, g2, b2, W3, g3, b3, W4, g4, b4, W5, g5, b5,
           Ws, lng, lnb, Wl1, bl1, g6, b6, Wl2, bl2, g7, b7, Wl3, bl3):
    f32 = jnp.float32
    x8 = jnp.concatenate([x, jnp.zeros((_B, 5, _N), f32)], axis=1)
    Wq1 = jnp.pad(W1[:, 0:3], ((0, 0), (0, 5)))
    Wf1 = jnp.pad(W1[:, 3:6], ((0, 0), (0, 5)))

    col = lambda v: v[:, None]
    row = lambda v: v[None, :]

    x1 = _attn_conv(x8, Wq1, Wf1, col(g1), col(b1))
    x2 = _attn_conv(x1, W2[:, 0:64], W2[:, 64:128], col(g2), col(b2))
    x3 = _attn_conv(x2, W3[:, 0:64], W3[:, 64:128], col(g3), col(b3))
    x4 = _attn_conv(x3, W4[:, 0:128], W4[:, 128:256], col(g4), col(b4))

    ap = _head(x1, x2, x3, x4, W5, col(g5), col(b5), Ws)
    ap = ap.reshape(_B, _HEADS * _EMB)

    return _mlp(ap, row(lng), row(lnb), Wl1, row(bl1), row(g6), row(b6),
                Wl2, row(bl2), row(g7), row(b7), Wl3, row(bl3))


# speedup vs baseline: 35.2080x; 1.7499x over previous
import functools
import math

import jax
import jax.numpy as jnp
from jax.experimental import pallas as pl
from jax.experimental.pallas import tpu as pltpu

_B, _N, _K, _EMB, _HEADS = 8, 1024, 20, 1024, 4
_EPS = 1e-5
_RB = 256
_NEG = float("-inf")
_SLOPE = 0.2


def _leaky(v):
    return jnp.where(v >= 0, v, _SLOPE * v)


def _attn_conv_body(xf_ref, xb_ref, wq_ref, wf_ref, g_ref, b_ref, out_ref):
    X = xf_ref[0]
    Xb = xb_ref[0]
    c = X.shape[0]

    G = jax.lax.dot_general(Xb, X, (((0,), (0,)), ((), ())),
                            preferred_element_type=jnp.float32)
    xx_row = jnp.sum(X * X, axis=0, keepdims=True)
    ones_c = jnp.ones((c, 1), jnp.float32)
    xx_col = jax.lax.dot_general(Xb * Xb, ones_c, (((0,), (0,)), ((), ())),
                                 preferred_element_type=jnp.float32)
    pd = 2.0 * G - xx_col - xx_row

    t = jnp.max(pd, axis=1, keepdims=True)
    selfmax = t
    for _ in range(_K):
        t = jnp.max(jnp.where(pd < t, pd, _NEG), axis=1, keepdims=True)
    member = jnp.logical_and(pd >= t, pd < selfmax)

    m_all = jnp.max(G, axis=1, keepdims=True)
    e = jnp.where(member, jnp.exp(G - m_all), 0.0)
    Zrow = jax.lax.dot_general(jnp.ones((1, pd.shape[1]), jnp.float32), e,
                               (((1,), (1,)), ((), ())),
                               preferred_element_type=jnp.float32)

    FT = jax.lax.dot_general(X, e, (((1,), (1,)), ((), ())),
                             preferred_element_type=jnp.float32)
    val = FT / Zrow - Xb

    Y = jax.lax.dot_general(wq_ref[...], Xb, (((1,), (0,)), ((), ())),
                            preferred_element_type=jnp.float32)
    Y = Y + jax.lax.dot_general(wf_ref[...], val, (((1,), (0,)), ((), ())),
                                preferred_element_type=jnp.float32)
    Y = Y / jnp.sqrt(jnp.float32(1.0 + _EPS))
    Y = Y * g_ref[...] + b_ref[...]
    out_ref[0] = _leaky(Y)


def _attn_conv(x, wq, wf, g, b):
    Bn, c, n = x.shape
    oc = wq.shape[0]
    grid = (Bn, n // _RB)
    return pl.pallas_call(
        _attn_conv_body,
        grid=grid,
        in_specs=[
            pl.BlockSpec((1, c, n), lambda bi, r: (bi, 0, 0)),
            pl.BlockSpec((1, c, _RB), lambda bi, r: (bi, 0, r)),
            pl.BlockSpec((oc, c), lambda bi, r: (0, 0)),
            pl.BlockSpec((oc, c), lambda bi, r: (0, 0)),
            pl.BlockSpec((oc, 1), lambda bi, r: (0, 0)),
            pl.BlockSpec((oc, 1), lambda bi, r: (0, 0)),
        ],
        out_specs=pl.BlockSpec((1, oc, _RB), lambda bi, r: (bi, 0, r)),
        out_shape=jax.ShapeDtypeStruct((Bn, oc, n), jnp.float32),
        compiler_params=pltpu.CompilerParams(
            dimension_semantics=("parallel", "parallel")),
    )(x, x, wq, wf, g, b)


def _head_body(x1_ref, x2_ref, x3_ref, x4_ref, w5_ref, g5_ref, b5_ref,
               ws_ref, ap_ref):
    W5 = w5_ref[...]
    h = jax.lax.dot_general(W5[:, 0:64], x1_ref[0], (((1,), (0,)), ((), ())),
                            preferred_element_type=jnp.float32)
    h = h + jax.lax.dot_general(W5[:, 64:128], x2_ref[0],
                                (((1,), (0,)), ((), ())),
                                preferred_element_type=jnp.float32)
    h = h + jax.lax.dot_general(W5[:, 128:256], x3_ref[0],
                                (((1,), (0,)), ((), ())),
                                preferred_element_type=jnp.float32)
    h = h + jax.lax.dot_general(W5[:, 256:512], x4_ref[0],
                                (((1,), (0,)), ((), ())),
                                preferred_element_type=jnp.float32)
    h = h / jnp.sqrt(jnp.float32(1.0 + _EPS))
    h = _leaky(h * g5_ref[...] + b5_ref[...])
    S = _leaky(jax.lax.dot_general(ws_ref[...], h, (((1,), (0,)), ((), ())),
                                   preferred_element_type=jnp.float32))
    ap_ref[0] = jax.lax.dot_general(S, h, (((1,), (1,)), ((), ())),
                                    preferred_element_type=jnp.float32)


def _head(x1, x2, x3, x4, W5, g5, b5, Ws):
    return pl.pallas_call(
        _head_body,
        grid=(_B,),
        in_specs=[
            pl.BlockSpec((1, 64, _N), lambda bi: (bi, 0, 0)),
            pl.BlockSpec((1, 64, _N), lambda bi: (bi, 0, 0)),
            pl.BlockSpec((1, 128, _N), lambda bi: (bi, 0, 0)),
            pl.BlockSpec((1, 256, _N), lambda bi: (bi, 0, 0)),
            pl.BlockSpec((_EMB, 512), lambda bi: (0, 0)),
            pl.BlockSpec((_EMB, 1), lambda bi: (0, 0)),
            pl.BlockSpec((_EMB, 1), lambda bi: (0, 0)),
            pl.BlockSpec((_HEADS, _EMB), lambda bi: (0, 0)),
        ],
        out_specs=pl.BlockSpec((1, _HEADS, _EMB), lambda bi: (bi, 0, 0)),
        out_shape=jax.ShapeDtypeStruct((_B, _HEADS, _EMB), jnp.float32),
        compiler_params=pltpu.CompilerParams(
            dimension_semantics=("parallel",)),
    )(x1, x2, x3, x4, W5, g5, b5, Ws)


def _mlp_body(ap_ref, lng_ref, lnb_ref, wl1_ref, bl1_ref, g6_ref, b6_ref,
              wl2_ref, bl2_ref, g7_ref, b7_ref, wl3_ref, bl3_ref, out_ref):
    ap = ap_ref[...]
    mu = jnp.mean(ap, axis=1, keepdims=True)
    d = ap - mu
    var = jnp.mean(d * d, axis=1, keepdims=True)
    ap = d / jnp.sqrt(var + _EPS) * lng_ref[...] + lnb_ref[...]
    ap = _leaky(ap)
    sq = jnp.sqrt(jnp.float32(1.0 + _EPS))
    y = jax.lax.dot_general(ap, wl1_ref[...], (((1,), (1,)), ((), ())),
                            preferred_element_type=jnp.float32) + bl1_ref[...]
    y = _leaky(y / sq * g6_ref[...] + b6_ref[...])
    y = jax.lax.dot_general(y, wl2_ref[...], (((1,), (1,)), ((), ())),
                            preferred_element_type=jnp.float32) + bl2_ref[...]
    y = _leaky(y / sq * g7_ref[...] + b7_ref[...])
    out_ref[...] = jax.lax.dot_general(
        y, wl3_ref[...], (((1,), (1,)), ((), ())),
        preferred_element_type=jnp.float32) + bl3_ref[...]


def _mlp(ap, lng, lnb, Wl1, bl1, g6, b6, Wl2, bl2, g7, b7, Wl3, bl3):
    return pl.pallas_call(
        _mlp_body,
        out_shape=jax.ShapeDtypeStruct((_B, 40), jnp.float32),
    )(ap, lng, lnb, Wl1, bl1, g6, b6, Wl2, bl2, g7, b7, Wl3, bl3)


def kernel(x, W1, g1, b1, W2, g2, b2, W3, g3, b3, W4, g4, b4, W5, g5, b5,
           Ws, lng, lnb, Wl1, bl1, g6, b6, Wl2, bl2, g7, b7, Wl3, bl3):
    f32 = jnp.float32
    x8 = jnp.concatenate([x, jnp.zeros((_B, 5, _N), f32)], axis=1)
    Wq1 = jnp.pad(W1[:, 0:3], ((0, 0), (0, 5)))
    Wf1 = jnp.pad(W1[:, 3:6], ((0, 0), (0, 5)))

    col = lambda v: v[:, None]
    row = lambda v: v[None, :]

    x1 = _attn_conv(x8, Wq1, Wf1, col(g1), col(b1))
    x2 = _attn_conv(x1, W2[:, 0:64], W2[:, 64:128], col(g2), col(b2))
    x3 = _attn_conv(x2, W3[:, 0:64], W3[:, 64:128], col(g3), col(b3))
    x4 = _attn_conv(x3, W4[:, 0:128], W4[:, 128:256], col(g4), col(b4))

    ap = _head(x1, x2, x3, x4, W5, col(g5), col(b5), Ws)
    ap = ap.reshape(_B, _HEADS * _EMB)

    return _mlp(ap, row(lng), row(lnb), Wl1, row(bl1), row(g6), row(b6),
                Wl2, row(bl2), row(g7), row(b7), Wl3, row(bl3))
